# Initial kernel scaffold; baseline (speedup 1.0000x reference)
import jax
import jax.numpy as jnp
from jax.experimental import pallas as pl


def kernel(x, gate_w, gate_b, w1, b1, w2, b2):
    raise NotImplementedError("write your pallas kernel here")


# speedup vs baseline: 3.7011x; 3.7011x over previous
import functools

import jax
import jax.numpy as jnp
from jax.experimental import pallas as pl
from jax.experimental.pallas import tpu as pltpu

_B, _S, _H = 1, 2048, 768
_E = 8
_FF = _H * 4
_FFB = 1024
_NFF = _FF // _FFB


def _moe_body(x_ref, gw_ref, gb_ref, w1_ref, b1_ref, w2_ref, b2_ref,
              out_ref, probs_ref):
    e = pl.program_id(0)
    f = pl.program_id(1)

    @pl.when((e == 0) & (f == 0))
    def _init():
        logits = jnp.dot(x_ref[...], gw_ref[...],
                         preferred_element_type=jnp.float32) + gb_ref[0]
        m = jnp.max(logits, axis=-1, keepdims=True)
        ex = jnp.exp(logits - m)
        p = ex / jnp.sum(ex, axis=-1, keepdims=True)
        p = p / jnp.sum(p, axis=-1, keepdims=True)
        probs_ref[...] = p
        out_ref[...] = jnp.dot(p, b2_ref[...],
                               preferred_element_type=jnp.float32)

    lane = jax.lax.broadcasted_iota(jnp.int32, (_S, _E), 1)
    pcol = jnp.sum(jnp.where(lane == e, probs_ref[...], 0.0),
                   axis=1, keepdims=True)

    h = jnp.dot(x_ref[...], w1_ref[0], preferred_element_type=jnp.float32)
    h = h + b1_ref[0, 0]
    h = 0.5 * h * (1.0 + jax.lax.erf(h * 0.7071067811865476))
    out_ref[...] += jnp.dot(h * pcol, w2_ref[0],
                            preferred_element_type=jnp.float32)


@jax.jit
def kernel(x, gate_w, gate_b, w1, b1, w2, b2):
    b, s, h = x.shape
    xf = x.reshape(-1, h)
    t = xf.shape[0]

    out = pl.pallas_call(
        _moe_body,
        grid=(_E, _NFF),
        in_specs=[
            pl.BlockSpec((t, h), lambda e, f: (0, 0)),
            pl.BlockSpec((h, _E), lambda e, f: (0, 0)),
            pl.BlockSpec((1, _E), lambda e, f: (0, 0)),
            pl.BlockSpec((1, h, _FFB), lambda e, f: (e, 0, f)),
            pl.BlockSpec((1, 1, _FFB), lambda e, f: (e, 0, f)),
            pl.BlockSpec((1, _FFB, h), lambda e, f: (e, f, 0)),
            pl.BlockSpec((_E, h), lambda e, f: (0, 0)),
        ],
        out_specs=pl.BlockSpec((t, h), lambda e, f: (0, 0)),
        out_shape=jax.ShapeDtypeStruct((t, h), jnp.float32),
        scratch_shapes=[pltpu.VMEM((t, _E), jnp.float32)],
        compiler_params=pltpu.CompilerParams(
            dimension_semantics=("arbitrary", "arbitrary"),
        ),
    )(xf, gate_w, gate_b.reshape(1, _E), w1, b1.reshape(_E, 1, _FF), w2, b2)
    return out.reshape(b, s, h)
